# Initial kernel scaffold; baseline (speedup 1.0000x reference)
import jax
import jax.numpy as jnp
from jax.experimental import pallas as pl


def kernel(x, edge_index, h_prev, B, Pi):
    raise NotImplementedError("write your pallas kernel here")


# speedup vs baseline: 2.2073x; 2.2073x over previous
import functools

import jax
import jax.numpy as jnp
from jax import lax
from jax.experimental import pallas as pl
from jax.experimental.pallas import tpu as pltpu
from jax.experimental.pallas import tpu_sc as plsc

NUM_SC_CORES = 2
NUM_SC_SUBCORES = 16
NUM_WORKERS = NUM_SC_CORES * NUM_SC_SUBCORES


def _tables_body(bt_ref, pi_ref, post_ref, ll_ref):
    bt = bt_ref[...]
    pi = pi_ref[...]
    sm_b = jax.nn.softmax(bt, axis=0)
    sm_pi = jax.nn.softmax(pi, axis=0)
    unnorm = sm_pi[None, :, :] * sm_b
    denom = jnp.sum(unnorm, axis=1)
    post_ref[...] = unnorm / denom[:, None, :]
    ll_ref[...] = jnp.log(denom)


def _ll_body(x_ref, tbl_ref, out_ref):
    xv = x_ref[0, 0, :]
    m = tbl_ref.shape[0]
    onehot = (xv[:, None] == lax.broadcasted_iota(jnp.int32, (xv.shape[0], m), 1))
    out_ref[...] = jnp.dot(onehot.astype(jnp.float32), tbl_ref[...],
                           preferred_element_type=jnp.float32)


def _sc_gather(table2d, idx, n, d, chunk):
    num_chunks = n // chunk
    iters = pl.cdiv(num_chunks, NUM_WORKERS)
    mesh = plsc.VectorSubcoreMesh(core_axis_name="c", subcore_axis_name="s")

    @functools.partial(
        pl.kernel,
        out_type=jax.ShapeDtypeStruct((n, d), jnp.float32),
        mesh=mesh,
        compiler_params=pltpu.CompilerParams(use_tc_tiling_on_sc=False),
        scratch_types=[
            pltpu.VMEM((chunk,), jnp.int32),
            pltpu.VMEM((chunk, d), jnp.float32),
            pltpu.SemaphoreType.DMA,
        ],
    )
    def gather_kernel(table_hbm, idx_hbm, out_hbm, idx_v, rows_v, sem):
        wid = lax.axis_index("s") * NUM_SC_CORES + lax.axis_index("c")

        @pl.loop(0, iters)
        def _(i):
            c = i * NUM_WORKERS + wid

            @pl.when(c < num_chunks)
            def _():
                base = c * chunk
                pltpu.sync_copy(idx_hbm.at[pl.ds(base, chunk)], idx_v)
                pltpu.async_copy(table_hbm.at[idx_v], rows_v, sem).wait()
                pltpu.sync_copy(rows_v, out_hbm.at[pl.ds(base, chunk)])

    return gather_kernel(table2d, idx)


def kernel(x, edge_index, h_prev, B, Pi):
    c, m, ngen = B.shape
    n = x.shape[0]
    d = c * ngen

    x = x.astype(jnp.int32)
    bt = jnp.transpose(B, (1, 0, 2))

    post3, ll_tbl = pl.pallas_call(
        _tables_body,
        out_shape=(
            jax.ShapeDtypeStruct((m, c, ngen), jnp.float32),
            jax.ShapeDtypeStruct((m, ngen), jnp.float32),
        ),
    )(bt, Pi)

    table2d = post3.reshape(m, d)

    chunk = 400
    assert n % chunk == 0 and chunk % 8 == 0
    post_flat = _sc_gather(table2d, x, n, d, chunk)

    bn = 1000
    assert n % bn == 0
    x3 = x.reshape(n // bn, 1, bn)
    ll = pl.pallas_call(
        _ll_body,
        grid=(n // bn,),
        in_specs=[
            pl.BlockSpec((1, 1, bn), lambda i: (i, 0, 0)),
            pl.BlockSpec((m, ngen), lambda i: (0, 0)),
        ],
        out_specs=pl.BlockSpec((bn, ngen), lambda i: (i, 0)),
        out_shape=jax.ShapeDtypeStruct((n, ngen), jnp.float32),
    )(x3, ll_tbl)

    return (ll.reshape(n, 1, ngen), post_flat.reshape(n, c, ngen))


# speedup vs baseline: 4.7502x; 2.1521x over previous
import functools

import jax
import jax.numpy as jnp
from jax import lax
from jax.experimental import pallas as pl
from jax.experimental.pallas import tpu as pltpu
from jax.experimental.pallas import tpu_sc as plsc

NUM_SC_CORES = 2
NUM_SC_SUBCORES = 16
NUM_WORKERS = NUM_SC_CORES * NUM_SC_SUBCORES


def _tables_body(bt_ref, pi_ref, post_ref, ll_ref):
    bt = bt_ref[...]
    pi = pi_ref[...]
    sm_b = jax.nn.softmax(bt, axis=0)
    sm_pi = jax.nn.softmax(pi, axis=0)
    unnorm = sm_pi[None, :, :] * sm_b
    denom = jnp.sum(unnorm, axis=1)
    post_ref[...] = unnorm / denom[:, None, :]
    ll_ref[...] = jnp.log(denom)


def _transpose_body(g_ref, out_ref, d: int):
    out_ref[...] = g_ref[...][:, :d].T


def _ll_body(x_ref, tbl_ref, out_ref, m: int):
    xv = x_ref[0, :]
    tbl = tbl_ref[...]
    acc = jnp.zeros(out_ref.shape, jnp.float32)
    for mm in range(m):
        sel = (xv == mm)[None, :]
        acc = jnp.where(sel, tbl[mm][:, None], acc)
    out_ref[...] = acc


def _sc_gather(table_pad, idx, n, dpad, chunk):
    num_chunks = n // chunk
    iters = pl.cdiv(num_chunks, NUM_WORKERS)
    mesh = plsc.VectorSubcoreMesh(core_axis_name="c", subcore_axis_name="s")

    @functools.partial(
        pl.kernel,
        out_type=jax.ShapeDtypeStruct((n, dpad), jnp.float32),
        mesh=mesh,
        scratch_types=[
            pltpu.VMEM((chunk,), jnp.int32),
            pltpu.VMEM((chunk, dpad), jnp.float32),
            pltpu.SemaphoreType.DMA,
        ],
    )
    def gather_kernel(table_hbm, idx_hbm, out_hbm, idx_v, rows_v, sem):
        wid = lax.axis_index("s") * NUM_SC_CORES + lax.axis_index("c")

        @pl.loop(0, iters)
        def _(i):
            c = i * NUM_WORKERS + wid

            @pl.when(c < num_chunks)
            def _():
                base = c * chunk
                pltpu.sync_copy(idx_hbm.at[pl.ds(base, chunk)], idx_v)
                pltpu.async_copy(table_hbm.at[idx_v], rows_v, sem).wait()
                pltpu.sync_copy(rows_v, out_hbm.at[pl.ds(base, chunk)])

    return gather_kernel(table_pad, idx)


def kernel(x, edge_index, h_prev, B, Pi):
    c, m, ngen = B.shape
    n = x.shape[0]
    d = c * ngen
    dpad = 2 * 128

    x = x.astype(jnp.int32)
    bt = jnp.transpose(B, (1, 0, 2))

    post3, ll_tbl = pl.pallas_call(
        _tables_body,
        out_shape=(
            jax.ShapeDtypeStruct((m, c, ngen), jnp.float32),
            jax.ShapeDtypeStruct((m, ngen), jnp.float32),
        ),
    )(bt, Pi)

    table_pad = jnp.pad(post3.reshape(m, d), ((0, 0), (0, dpad - d)))

    chunk = 400
    assert n % chunk == 0 and chunk % 8 == 0
    gp = _sc_gather(table_pad, x, n, dpad, chunk)

    bn = 2048
    grid = pl.cdiv(n, bn)
    p2d = pl.pallas_call(
        functools.partial(_transpose_body, d=d),
        grid=(grid,),
        in_specs=[pl.BlockSpec((bn, dpad), lambda i: (i, 0))],
        out_specs=pl.BlockSpec((d, bn), lambda i: (0, i)),
        out_shape=jax.ShapeDtypeStruct((d, n), jnp.float32),
    )(gp)

    x2 = x.reshape(1, n)
    l2d = pl.pallas_call(
        functools.partial(_ll_body, m=m),
        grid=(grid,),
        in_specs=[
            pl.BlockSpec((1, bn), lambda i: (0, i)),
            pl.BlockSpec((m, ngen), lambda i: (0, 0)),
        ],
        out_specs=pl.BlockSpec((ngen, bn), lambda i: (0, i)),
        out_shape=jax.ShapeDtypeStruct((ngen, n), jnp.float32),
    )(x2, ll_tbl)

    log_likelihood = jnp.transpose(l2d, (1, 0))[:, None, :]
    posterior = jnp.transpose(p2d.reshape(c, ngen, n), (2, 0, 1))
    return (log_likelihood, posterior)
